# baseline (speedup 1.0000x reference)
import functools

import jax
import jax.numpy as jnp
from jax import lax
from jax.experimental import pallas as pl
from jax.experimental.pallas import tpu as pltpu
from jax.experimental.pallas import tpu_sc as plsc

_B = 8
_C = 21
_H = 512
_W = 512

_NW = 32
_WPB = _NW // _B
_ROWS_PER_W = _H // _WPB
_CR = 4
_NCHUNK = _ROWS_PER_W // _CR
_GPR = _W // 16


def _make_segmax():
    mesh = plsc.VectorSubcoreMesh(core_axis_name="c", subcore_axis_name="s")

    @functools.partial(
        pl.kernel,
        mesh=mesh,
        out_type=jax.ShapeDtypeStruct((_NW, 32), jnp.float32),
        compiler_params=pltpu.CompilerParams(needs_layout_passes=False),
        scratch_types=[
            pltpu.VMEM((_C, _CR, _W), jnp.float32),
            pltpu.VMEM((_C, _CR, _W), jnp.float32),
            pltpu.VMEM((16, 32), jnp.float32),
            pltpu.VMEM((32,), jnp.float32),
            pltpu.SemaphoreType.DMA,
            pltpu.SemaphoreType.DMA,
        ],
    )
    def segmax(x_hbm, out_hbm, v0, v1, acc2, acc, sem0, sem1):
        wid = lax.axis_index("s") * 2 + lax.axis_index("c")
        b = wid // _WPB
        row0 = (wid % _WPB) * _ROWS_PER_W
        zeros = jnp.zeros((16,), jnp.float32)
        for r in range(16):
            acc2[r, pl.ds(0, 16)] = zeros
            acc2[r, pl.ds(16, 16)] = zeros
        rows16 = lax.iota(jnp.int32, 16)

        def start(chunk, buf, sem):
            r = row0 + chunk * _CR
            pltpu.make_async_copy(
                x_hbm.at[b, :, pl.ds(r, _CR), :], buf, sem
            ).start()

        def drain(buf, sem):
            pltpu.make_async_copy(
                x_hbm.at[b, :, pl.ds(row0, _CR), :], buf, sem
            ).wait()

        def compute(buf):
            def row_body(r, carry):
                for g in range(_GPR):
                    sl = pl.ds(g * 16, 16)
                    m = buf[0, r, sl]
                    a = jnp.zeros((16,), jnp.int32)
                    for c in range(1, _C):
                        xc = buf[c, r, sl]
                        upd = xc > m
                        m = jnp.where(upd, xc, m)
                        a = jnp.where(upd, c, a)
                    plsc.addupdate_scatter(acc2, [rows16, a], m)
                return carry

            lax.fori_loop(0, _CR, row_body, 0)

        start(0, v0, sem0)

        def pair_body(p, carry):
            start(2 * p + 1, v1, sem1)
            drain(v0, sem0)
            compute(v0)

            @pl.when(p < _NCHUNK // 2 - 1)
            def _():
                start(2 * p + 2, v0, sem0)

            drain(v1, sem1)
            compute(v1)
            return carry

        lax.fori_loop(0, _NCHUNK // 2, pair_body, 0)

        lo = acc2[0, pl.ds(0, 16)]
        hi = acc2[0, pl.ds(16, 16)]
        for r in range(1, 16):
            lo = lo + acc2[r, pl.ds(0, 16)]
            hi = hi + acc2[r, pl.ds(16, 16)]
        acc[pl.ds(0, 16)] = lo
        acc[pl.ds(16, 16)] = hi
        pltpu.sync_copy(acc, out_hbm.at[wid])

    return segmax


@functools.cache
def _segmax():
    return _make_segmax()


def kernel(x):
    partials = _segmax()(x)
    return partials.reshape(_B, _WPB, 32).sum(axis=1)[:, :_C]


# speedup vs baseline: 3.8101x; 3.8101x over previous
import functools

import jax
import jax.numpy as jnp
from jax import lax
from jax.experimental import pallas as pl
from jax.experimental.pallas import tpu as pltpu
from jax.experimental.pallas import tpu_sc as plsc

_B = 8
_C = 21
_H = 512
_W = 512
_R = 128

_NW = 32
_PROWS = _B * _H
_ROWS_PER_W = _PROWS // _NW
_SUBROWS = 16
_UNROLL = 8


def _maxarg_body(x_ref, p_ref):
    x = x_ref[0]
    m = x[0]
    a = jnp.zeros(m.shape, jnp.int32)
    for c in range(1, _C):
        xc = x[c]
        upd = xc > m
        m = jnp.where(upd, xc, m)
        a = jnp.where(upd, c, a)
    mi = lax.bitcast_convert_type(m, jnp.int32)
    p_ref[0] = (mi & -32) | a


def _stage1(x):
    return pl.pallas_call(
        _maxarg_body,
        grid=(_B, _H // _R),
        in_specs=[pl.BlockSpec((1, _C, _R, _W), lambda b, t: (b, 0, t, 0))],
        out_specs=pl.BlockSpec((1, _R, _W), lambda b, t: (b, t, 0)),
        out_shape=jax.ShapeDtypeStruct((_B, _H, _W), jnp.int32),
    )(x)


def _make_stage2():
    mesh = plsc.VectorSubcoreMesh(core_axis_name="c", subcore_axis_name="s")

    @functools.partial(
        pl.kernel,
        mesh=mesh,
        out_type=jax.ShapeDtypeStruct((_NW, 32), jnp.float32),
        compiler_params=pltpu.CompilerParams(needs_layout_passes=False),
        scratch_types=[
            pltpu.VMEM((_SUBROWS, _W), jnp.int32),
            pltpu.VMEM((_SUBROWS, _W), jnp.int32),
            pltpu.VMEM((16, 32), jnp.float32),
            pltpu.VMEM((32,), jnp.float32),
            pltpu.SemaphoreType.DMA,
            pltpu.SemaphoreType.DMA,
        ],
    )
    def segsum(p_hbm, out_hbm, pv0, pv1, acc2, acc, sem0, sem1):
        wid = lax.axis_index("s") * 2 + lax.axis_index("c")
        base = wid * _ROWS_PER_W
        bufs = (pv0, pv1)
        sems = (sem0, sem1)
        zeros = jnp.zeros((16,), jnp.float32)
        for r in range(16):
            acc2[r, pl.ds(0, 16)] = zeros
            acc2[r, pl.ds(16, 16)] = zeros
        rows = lax.iota(jnp.int32, 16)
        n_sub = _ROWS_PER_W // _SUBROWS

        copies = [
            pltpu.make_async_copy(
                p_hbm.at[pl.ds(base + s * _SUBROWS, _SUBROWS), :],
                bufs[s % 2],
                sems[s % 2],
            )
            for s in range(n_sub)
        ]
        copies[0].start()
        for sub in range(n_sub):
            if sub + 1 < n_sub:
                copies[sub + 1].start()
            copies[sub].wait()
            buf = bufs[sub % 2]

            def body(r, carry, buf=buf):
                for u in range(_W // 16):
                    p = buf[r, pl.ds(u * 16, 16)]
                    ids = p & 31
                    vals = plsc.bitcast(p & -32, jnp.float32)
                    plsc.addupdate_scatter(acc2, [rows, ids], vals)
                return carry

            lax.fori_loop(0, _SUBROWS, body, 0)

        lo = acc2[0, pl.ds(0, 16)]
        hi = acc2[0, pl.ds(16, 16)]
        for r in range(1, 16):
            lo = lo + acc2[r, pl.ds(0, 16)]
            hi = hi + acc2[r, pl.ds(16, 16)]
        acc[pl.ds(0, 16)] = lo
        acc[pl.ds(16, 16)] = hi
        pltpu.sync_copy(acc, out_hbm.at[wid])

    return segsum


@functools.cache
def _stage2():
    return _make_stage2()


def kernel(x):
    p = _stage1(x)
    partials = _stage2()(p.reshape(_PROWS, _W))
    return partials.reshape(_B, _NW // _B, 32).sum(axis=1)[:, :_C]
